# Initial kernel scaffold; baseline (speedup 1.0000x reference)
import functools

import jax
import jax.numpy as jnp
from jax import lax
from jax.experimental import pallas as pl
from jax.experimental.pallas import tpu as pltpu
from jax.experimental.pallas import tpu_sc as plsc

KNN = 16
SLOPE = 0.1
QT = 256
QM = 256

_NC, _NS = 2, 16
_NW = _NC * _NS
_CHUNK = 128


def _dot(a, b):
    return lax.dot_general(a, b, (((1,), (0,)), ((), ())),
                           preferred_element_type=jnp.float32,
                           precision=lax.Precision.HIGHEST)


def _leaky(x):
    return jnp.where(x >= 0, x, SLOPE * x)


def _stage0_body(featA, featB, qxp, kxp, Wt11T, bt11, Wt22T, bt22, WdT, bd,
                 WposT, bpos, qadd, ktab, qf, kf):
    fA = featA[0]
    fB = featB[0]
    xq = qxp[0]
    xk = kxp[0]
    qfeat = _dot(fA, Wt11T[...]) + bt11[...]
    kfeat = _dot(fB, Wt22T[...]) + bt22[...]
    qadd[0] = qfeat - _dot(xq, WposT[...]) + bpos[...]
    ktab[0] = kfeat + _dot(xk, WposT[...])
    f1 = _dot(qfeat, WdT[...]) + bd[...]
    f2 = _dot(kfeat, WdT[...]) + bd[...]
    n1 = jnp.sqrt(jnp.sum(f1 * f1, axis=1, keepdims=True))
    n2 = jnp.sqrt(jnp.sum(f2 * f2, axis=1, keepdims=True))
    qf[0] = f1 / (n1 + 1e-8)
    kf[0] = f2 / (n2 + 1e-8)


def _stage1_body(qf, kf, qxp, kxp, idx_out, *, n_keys):
    cb = pl.program_id(0)
    a = qf[0]
    b = kf[0]
    xq = qxp[0]
    xk = kxp[0]
    qsq = jnp.sum(xq * xq, axis=1, keepdims=True)
    ksq = jnp.sum(xk * xk, axis=1, keepdims=True)
    ones_q = jnp.ones_like(qsq)
    ones_k = jnp.ones_like(ksq)
    A = jnp.concatenate([-a, xq * -2.0, ones_q, qsq + 1.0], axis=1)
    Bm = jnp.concatenate([b, xk, ksq, ones_k], axis=1)
    dist = lax.dot_general(A, Bm, (((1,), (1,)), ((), ())),
                           preferred_element_type=jnp.float32,
                           precision=lax.Precision.HIGHEST)
    iota = lax.broadcasted_iota(jnp.int32, dist.shape, 1)
    big = jnp.int32(n_keys)
    idxs = []
    d = dist
    for _ in range(KNN):
        m = jnp.min(d, axis=1, keepdims=True)
        iv = jnp.min(jnp.where(d == m, iota, big), axis=1, keepdims=True)
        idxs.append(iv)
        d = jnp.where(iota == iv, jnp.inf, d)
    idx_out[0] = jnp.concatenate(idxs, axis=1) + cb * n_keys


def _stage3_body(g, qadd, Wm1T, bm1, Wm2T, bm2, out):
    gg = g[0]
    q = qadd[0]
    m0 = q.shape[1]
    h = gg.reshape(QM, KNN, m0) + q[:, None, :]
    h = _leaky(h).reshape(QM * KNN, m0)
    h = _leaky(_dot(h, Wm1T[...]) + bm1[...])
    h = _leaky(_dot(h, Wm2T[...]) + bm2[...])
    out[0] = jnp.max(h.reshape(QM, KNN, out.shape[2]), axis=1)


def _sc_gather(tbl, idx):
    total = idx.shape[0]
    d = tbl.shape[1]
    per_w = total // _NW
    steps = per_w // _CHUNK
    mesh = plsc.VectorSubcoreMesh(core_axis_name="c", subcore_axis_name="s")

    @functools.partial(
        pl.kernel,
        out_type=jax.ShapeDtypeStruct((total, d), jnp.float32),
        mesh=mesh,
        scratch_types=[
            pltpu.VMEM((2, _CHUNK), jnp.int32),
            pltpu.VMEM((2, _CHUNK, d), jnp.float32),
            pltpu.SemaphoreType.DMA,
            pltpu.SemaphoreType.DMA,
        ],
    )
    def gk(idx_hbm, tbl_hbm, out_hbm, idx_v, rows_v, sem0, sem1):
        wid = lax.axis_index("s") * _NC + lax.axis_index("c")
        base = wid * per_w
        sems = (sem0, sem1)

        def start(j, slot):
            off = base + j * _CHUNK
            pltpu.sync_copy(idx_hbm.at[pl.ds(off, _CHUNK)], idx_v.at[slot])
            return pltpu.async_copy(tbl_hbm.at[idx_v.at[slot]],
                                    rows_v.at[slot], sems[slot])

        def drain(j, slot, cp):
            off = base + j * _CHUNK
            cp.wait()
            pltpu.sync_copy(rows_v.at[slot], out_hbm.at[pl.ds(off, _CHUNK)])

        def step(jj, carry):
            j = jj * 2
            c0 = start(j, 0)
            c1 = start(j + 1, 1)
            drain(j, 0, c0)
            drain(j + 1, 1, c1)
            return carry

        lax.fori_loop(0, steps // 2, step, 0)

    return gk(idx, tbl)


def kernel(pc1, pc2, feat1, feat2, Wt11, bt11, Wt22, bt22, Wpos, bpos,
           Wd, bd, Wm1, bm1, Wm2, bm2):
    B = pc1.shape[0]
    N = pc1.shape[2]
    C = feat1.shape[1]
    M0 = Wt11.shape[0]
    DO = Wd.shape[0]
    OUT = Wm2.shape[0]
    CB = 2 * B
    f32 = jnp.float32

    f1t = jnp.transpose(feat1, (0, 2, 1))
    f2t = jnp.transpose(feat2, (0, 2, 1))
    featA = jnp.concatenate([f1t, f2t], axis=0)
    featB = jnp.concatenate([f2t, f1t], axis=0)
    x1t = jnp.transpose(pc1, (0, 2, 1))
    x2t = jnp.transpose(pc2, (0, 2, 1))
    qxyz = jnp.concatenate([x1t, x2t], axis=0)
    kxyz = jnp.concatenate([x2t, x1t], axis=0)
    qxp = jnp.pad(qxyz, ((0, 0), (0, 0), (0, 13)))
    kxp = jnp.pad(kxyz, ((0, 0), (0, 0), (0, 13)))

    WposT = jnp.pad(Wpos.T, ((0, 13), (0, 0)))
    wfull = lambda shp: pl.BlockSpec(shp, lambda *a: tuple(0 for _ in shp))

    cbN = lambda d: pl.BlockSpec((1, N, d), lambda cb: (cb, 0, 0))
    qadd, ktab, qf, kf = pl.pallas_call(
        _stage0_body,
        grid=(CB,),
        in_specs=[cbN(C), cbN(C), cbN(16), cbN(16),
                  wfull((C, M0)), wfull((1, M0)), wfull((C, M0)), wfull((1, M0)),
                  wfull((M0, DO)), wfull((1, DO)), wfull((16, M0)), wfull((1, M0))],
        out_specs=[cbN(M0), cbN(M0), cbN(DO), cbN(DO)],
        out_shape=[jax.ShapeDtypeStruct((CB, N, M0), f32),
                   jax.ShapeDtypeStruct((CB, N, M0), f32),
                   jax.ShapeDtypeStruct((CB, N, DO), f32),
                   jax.ShapeDtypeStruct((CB, N, DO), f32)],
    )(featA, featB, qxp, kxp,
      Wt11.T, bt11[None, :], Wt22.T, bt22[None, :],
      Wd.T, bd[None, :], WposT, bpos[None, :])

    qblk = lambda d: pl.BlockSpec((1, QT, d), lambda cb, t: (cb, t, 0))
    kblk = lambda d: pl.BlockSpec((1, N, d), lambda cb, t: (cb, 0, 0))
    idxg = pl.pallas_call(
        functools.partial(_stage1_body, n_keys=N),
        grid=(CB, N // QT),
        in_specs=[qblk(DO), kblk(DO), qblk(16), kblk(16)],
        out_specs=pl.BlockSpec((1, QT, KNN), lambda cb, t: (cb, t, 0)),
        out_shape=jax.ShapeDtypeStruct((CB, N, KNN), jnp.int32),
    )(qf, kf, qxp, kxp)

    g = _sc_gather(ktab.reshape(CB * N, M0), idxg.reshape(CB * N * KNN))

    out = pl.pallas_call(
        _stage3_body,
        grid=(CB, N // QM),
        in_specs=[pl.BlockSpec((1, QM * KNN, M0), lambda cb, t: (cb, t, 0)),
                  pl.BlockSpec((1, QM, M0), lambda cb, t: (cb, t, 0)),
                  wfull((M0, M0)), wfull((1, M0)),
                  wfull((M0, OUT)), wfull((1, OUT))],
        out_specs=pl.BlockSpec((1, QM, OUT), lambda cb, t: (cb, t, 0)),
        out_shape=jax.ShapeDtypeStruct((CB, N, OUT), f32),
    )(g.reshape(CB, N * KNN, M0), qadd,
      Wm1.T, bm1[None, :], Wm2.T, bm2[None, :])

    feat1_new = jnp.transpose(out[:B], (0, 2, 1))
    feat2_new = jnp.transpose(out[B:], (0, 2, 1))
    return feat1_new, feat2_new


# speedup vs baseline: 15.8922x; 15.8922x over previous
import functools

import jax
import jax.numpy as jnp
from jax import lax
from jax.experimental import pallas as pl
from jax.experimental.pallas import tpu as pltpu
from jax.experimental.pallas import tpu_sc as plsc

KNN = 16
SLOPE = 0.1
QT = 256
QM = 256

_NC, _NS = 2, 16
_NW = _NC * _NS
_CHUNK = 128


def _dot(a, b):
    return lax.dot_general(a.astype(jnp.bfloat16), b.astype(jnp.bfloat16),
                           (((1,), (0,)), ((), ())),
                           preferred_element_type=jnp.float32)


def _leaky(x):
    return jnp.where(x >= 0, x, SLOPE * x)


def _stage0_body(featA, featB, qxp, kxp, Wt11T, bt11, Wt22T, bt22, WdT, bd,
                 WposT, bpos, qadd, ktab, qf, kf):
    fA = featA[0]
    fB = featB[0]
    xq = qxp[0]
    xk = kxp[0]
    qfeat = _dot(fA, Wt11T[...]) + bt11[...]
    kfeat = _dot(fB, Wt22T[...]) + bt22[...]
    qadd[0] = qfeat - _dot(xq, WposT[...]) + bpos[...]
    ktab[0] = kfeat + _dot(xk, WposT[...])
    f1 = _dot(qfeat, WdT[...]) + bd[...]
    f2 = _dot(kfeat, WdT[...]) + bd[...]
    n1 = jnp.sqrt(jnp.sum(f1 * f1, axis=1, keepdims=True))
    n2 = jnp.sqrt(jnp.sum(f2 * f2, axis=1, keepdims=True))
    qf[0] = f1 / (n1 + 1e-8)
    kf[0] = f2 / (n2 + 1e-8)


def _stage1_body(qf, kf, qxp, kxp, kxpT, idx_out, *, n_keys):
    cb = pl.program_id(0)
    a = qf[0]
    b = kf[0]
    xq = qxp[0]
    xk = kxp[0]
    xkT = kxpT[0]
    qsq = jnp.sum(xq * xq, axis=1, keepdims=True)
    ksq = jnp.sum(xkT * xkT, axis=0, keepdims=True)
    A = jnp.concatenate([-a, xq * -2.0], axis=1)
    Bm = jnp.concatenate([b, xk], axis=1)
    dist = lax.dot_general(A.astype(jnp.bfloat16), Bm.astype(jnp.bfloat16),
                           (((1,), (1,)), ((), ())),
                           preferred_element_type=jnp.float32)
    dist = dist + qsq + ksq + 1.0
    iota = lax.broadcasted_iota(jnp.int32, dist.shape, 1)
    big = jnp.int32(n_keys)
    idxs = []
    d = dist
    for _ in range(KNN):
        m = jnp.min(d, axis=1, keepdims=True)
        iv = jnp.min(jnp.where(d == m, iota, big), axis=1, keepdims=True)
        idxs.append(iv)
        d = jnp.where(iota == iv, jnp.inf, d)
    idx_out[0] = jnp.concatenate(idxs, axis=1) + cb * n_keys


def _stage3_body(g, qadd, Wm1T, bm1, Wm2T, bm2, out):
    gg = g[0]
    q = qadd[0]
    m0 = q.shape[1]
    h = gg.reshape(QM, KNN, m0) + q[:, None, :]
    h = _leaky(h).reshape(QM * KNN, m0)
    h = _leaky(_dot(h, Wm1T[...]) + bm1[...])
    h = _leaky(_dot(h, Wm2T[...]) + bm2[...])
    out[0] = jnp.max(h.reshape(QM, KNN, out.shape[2]), axis=1)


def _sc_gather(tbl, idx):
    total = idx.shape[0]
    d = tbl.shape[1]
    per_w = total // _NW
    steps = per_w // _CHUNK
    mesh = plsc.VectorSubcoreMesh(core_axis_name="c", subcore_axis_name="s")

    @functools.partial(
        pl.kernel,
        out_type=jax.ShapeDtypeStruct((total, d), jnp.float32),
        mesh=mesh,
        scratch_types=[
            pltpu.VMEM((2, _CHUNK), jnp.int32),
            pltpu.VMEM((2, _CHUNK, d), jnp.float32),
            pltpu.SemaphoreType.DMA,
            pltpu.SemaphoreType.DMA,
        ],
        compiler_params=pltpu.CompilerParams(use_tc_tiling_on_sc=False),
    )
    def gk(idx_hbm, tbl_hbm, out_hbm, idx_v, rows_v, sem0, sem1):
        wid = lax.axis_index("s") * _NC + lax.axis_index("c")
        base = wid * per_w
        sems = (sem0, sem1)

        def start(j, slot):
            off = base + j * _CHUNK
            pltpu.sync_copy(idx_hbm.at[pl.ds(off, _CHUNK)], idx_v.at[slot])
            return pltpu.async_copy(tbl_hbm.at[idx_v.at[slot]],
                                    rows_v.at[slot], sems[slot])

        def drain(j, slot, cp):
            off = base + j * _CHUNK
            cp.wait()
            pltpu.sync_copy(rows_v.at[slot], out_hbm.at[pl.ds(off, _CHUNK)])

        def step(jj, carry):
            j = jj * 2
            c0 = start(j, 0)
            c1 = start(j + 1, 1)
            drain(j, 0, c0)
            drain(j + 1, 1, c1)
            return carry

        lax.fori_loop(0, steps // 2, step, 0)

    return gk(idx, tbl)


def kernel(pc1, pc2, feat1, feat2, Wt11, bt11, Wt22, bt22, Wpos, bpos,
           Wd, bd, Wm1, bm1, Wm2, bm2):
    B = pc1.shape[0]
    N = pc1.shape[2]
    C = feat1.shape[1]
    M0 = Wt11.shape[0]
    DO = Wd.shape[0]
    OUT = Wm2.shape[0]
    CB = 2 * B
    f32 = jnp.float32

    f1t = jnp.transpose(feat1, (0, 2, 1))
    f2t = jnp.transpose(feat2, (0, 2, 1))
    featA = jnp.concatenate([f1t, f2t], axis=0)
    featB = jnp.concatenate([f2t, f1t], axis=0)
    x1t = jnp.transpose(pc1, (0, 2, 1))
    x2t = jnp.transpose(pc2, (0, 2, 1))
    qxyz = jnp.concatenate([x1t, x2t], axis=0)
    kxyz = jnp.concatenate([x2t, x1t], axis=0)
    qxp = jnp.pad(qxyz, ((0, 0), (0, 0), (0, 13)))
    kxp = jnp.pad(kxyz, ((0, 0), (0, 0), (0, 13)))

    WposT = jnp.pad(Wpos.T, ((0, 13), (0, 0)))
    wfull = lambda shp: pl.BlockSpec(shp, lambda *a: tuple(0 for _ in shp))

    S0T = 1024
    cbN = lambda d: pl.BlockSpec((1, S0T, d), lambda cb, t: (cb, t, 0))
    qadd, ktab, qf, kf = pl.pallas_call(
        _stage0_body,
        grid=(CB, N // S0T),
        in_specs=[cbN(C), cbN(C), cbN(16), cbN(16),
                  wfull((C, M0)), wfull((1, M0)), wfull((C, M0)), wfull((1, M0)),
                  wfull((M0, DO)), wfull((1, DO)), wfull((16, M0)), wfull((1, M0))],
        out_specs=[cbN(M0), cbN(M0), cbN(DO), cbN(DO)],
        out_shape=[jax.ShapeDtypeStruct((CB, N, M0), f32),
                   jax.ShapeDtypeStruct((CB, N, M0), f32),
                   jax.ShapeDtypeStruct((CB, N, DO), f32),
                   jax.ShapeDtypeStruct((CB, N, DO), f32)],
    )(featA, featB, qxp, kxp,
      Wt11.T, bt11[None, :], Wt22.T, bt22[None, :],
      Wd.T, bd[None, :], WposT, bpos[None, :])

    qblk = lambda d: pl.BlockSpec((1, QT, d), lambda cb, t: (cb, t, 0))
    kblk = lambda d: pl.BlockSpec((1, N, d), lambda cb, t: (cb, 0, 0))
    kxpT = jnp.transpose(kxp, (0, 2, 1))
    idxg = pl.pallas_call(
        functools.partial(_stage1_body, n_keys=N),
        grid=(CB, N // QT),
        in_specs=[qblk(DO), kblk(DO), qblk(16), kblk(16),
                  pl.BlockSpec((1, 16, N), lambda cb, t: (cb, 0, 0))],
        out_specs=pl.BlockSpec((1, QT, KNN), lambda cb, t: (cb, t, 0)),
        out_shape=jax.ShapeDtypeStruct((CB, N, KNN), jnp.int32),
    )(qf, kf, qxp, kxp, kxpT)

    g = _sc_gather(ktab.reshape(CB * N, M0), idxg.reshape(CB * N * KNN))

    out = pl.pallas_call(
        _stage3_body,
        grid=(CB, N // QM),
        in_specs=[pl.BlockSpec((1, QM * KNN, M0), lambda cb, t: (cb, t, 0)),
                  pl.BlockSpec((1, QM, M0), lambda cb, t: (cb, t, 0)),
                  wfull((M0, M0)), wfull((1, M0)),
                  wfull((M0, OUT)), wfull((1, OUT))],
        out_specs=pl.BlockSpec((1, QM, OUT), lambda cb, t: (cb, t, 0)),
        out_shape=jax.ShapeDtypeStruct((CB, N, OUT), f32),
    )(g.reshape(CB, N * KNN, M0), qadd,
      Wm1.T, bm1[None, :], Wm2.T, bm2[None, :])

    feat1_new = jnp.transpose(out[:B], (0, 2, 1))
    feat2_new = jnp.transpose(out[B:], (0, 2, 1))
    return feat1_new, feat2_new


# speedup vs baseline: 18.5071x; 1.1645x over previous
import functools

import jax
import jax.numpy as jnp
from jax import lax
from jax.experimental import pallas as pl
from jax.experimental.pallas import tpu as pltpu
from jax.experimental.pallas import tpu_sc as plsc

KNN = 16
SLOPE = 0.1
QT = 512
QM = 256

_NC, _NS = 2, 16
_NW = _NC * _NS
_CHUNK = 128


def _dot(a, b):
    return lax.dot_general(a.astype(jnp.bfloat16), b.astype(jnp.bfloat16),
                           (((1,), (0,)), ((), ())),
                           preferred_element_type=jnp.float32)


def _leaky(x):
    return jnp.where(x >= 0, x, SLOPE * x)


def _stage0_body(featA, featB, qxp, kxp, Wt11T, bt11, Wt22T, bt22, WdT, bd,
                 WposT, bpos, qadd, ktab, qf, kf):
    fA = featA[0]
    fB = featB[0]
    xq = qxp[0]
    xk = kxp[0]
    qfeat = _dot(fA, Wt11T[...]) + bt11[...]
    kfeat = _dot(fB, Wt22T[...]) + bt22[...]
    qadd[0] = qfeat - _dot(xq, WposT[...]) + bpos[...]
    ktab[0] = kfeat + _dot(xk, WposT[...])
    f1 = _dot(qfeat, WdT[...]) + bd[...]
    f2 = _dot(kfeat, WdT[...]) + bd[...]
    n1 = jnp.sqrt(jnp.sum(f1 * f1, axis=1, keepdims=True))
    n2 = jnp.sqrt(jnp.sum(f2 * f2, axis=1, keepdims=True))
    qf[0] = f1 / (n1 + 1e-8)
    kf[0] = f2 / (n2 + 1e-8)


def _stage1_body(qf, kf, qxp, kxp, kxpT, idx_out, *, n_keys):
    cb = pl.program_id(0)
    a = qf[0]
    b = kf[0]
    xq = qxp[0]
    xk = kxp[0]
    xkT = kxpT[0]
    qsq = jnp.sum(xq * xq, axis=1, keepdims=True)
    ksq = jnp.sum(xkT * xkT, axis=0, keepdims=True)
    A = jnp.concatenate([-a, xq * -2.0], axis=1)
    Bm = jnp.concatenate([b, xk], axis=1)
    dist = lax.dot_general(A.astype(jnp.bfloat16), Bm.astype(jnp.bfloat16),
                           (((1,), (1,)), ((), ())),
                           preferred_element_type=jnp.float32)
    dist = dist + qsq + ksq + 1.0
    iotaf = lax.broadcasted_iota(jnp.int32, dist.shape, 1).astype(jnp.float32)
    bigf = jnp.float32(n_keys)
    inf = jnp.float32(jnp.inf)
    idxs = []
    d = dist
    for _ in range(KNN):
        m = jnp.min(d, axis=1, keepdims=True)
        cand = jnp.where(d == m, iotaf, bigf)
        iv = jnp.min(cand, axis=1, keepdims=True)
        idxs.append(iv)
        d = jnp.where(cand == iv, inf, d)
    idx_out[0] = (jnp.concatenate(idxs, axis=1).astype(jnp.int32)
                  + cb * n_keys)


def _stage3_body(g, qadd, Wm1T, bm1, Wm2T, bm2, out):
    gg = g[0]
    q = qadd[0]
    m0 = q.shape[1]
    h = gg.reshape(QM, KNN, m0) + q[:, None, :]
    h = _leaky(h).reshape(QM * KNN, m0)
    h = _leaky(_dot(h, Wm1T[...]) + bm1[...])
    h = _leaky(_dot(h, Wm2T[...]) + bm2[...])
    out[0] = jnp.max(h.reshape(QM, KNN, out.shape[2]), axis=1)


def _sc_gather(tbl, idx):
    total = idx.shape[0]
    d = tbl.shape[1]
    per_w = total // _NW
    steps = per_w // _CHUNK
    mesh = plsc.VectorSubcoreMesh(core_axis_name="c", subcore_axis_name="s")

    @functools.partial(
        pl.kernel,
        out_type=jax.ShapeDtypeStruct((total, d), jnp.float32),
        mesh=mesh,
        scratch_types=[
            pltpu.VMEM((2, _CHUNK), jnp.int32),
            pltpu.VMEM((2, _CHUNK, d), jnp.float32),
            pltpu.SemaphoreType.DMA,
            pltpu.SemaphoreType.DMA,
        ],
        compiler_params=pltpu.CompilerParams(use_tc_tiling_on_sc=False),
    )
    def gk(idx_hbm, tbl_hbm, out_hbm, idx_v, rows_v, sem0, sem1):
        wid = lax.axis_index("s") * _NC + lax.axis_index("c")
        base = wid * per_w
        sems = (sem0, sem1)

        def start(j, slot):
            off = base + j * _CHUNK
            pltpu.sync_copy(idx_hbm.at[pl.ds(off, _CHUNK)], idx_v.at[slot])
            return pltpu.async_copy(tbl_hbm.at[idx_v.at[slot]],
                                    rows_v.at[slot], sems[slot])

        def drain(j, slot, cp):
            off = base + j * _CHUNK
            cp.wait()
            pltpu.sync_copy(rows_v.at[slot], out_hbm.at[pl.ds(off, _CHUNK)])

        def step(jj, carry):
            j = jj * 2
            c0 = start(j, 0)
            c1 = start(j + 1, 1)
            drain(j, 0, c0)
            drain(j + 1, 1, c1)
            return carry

        lax.fori_loop(0, steps // 2, step, 0)

    return gk(idx, tbl)


def kernel(pc1, pc2, feat1, feat2, Wt11, bt11, Wt22, bt22, Wpos, bpos,
           Wd, bd, Wm1, bm1, Wm2, bm2):
    B = pc1.shape[0]
    N = pc1.shape[2]
    C = feat1.shape[1]
    M0 = Wt11.shape[0]
    DO = Wd.shape[0]
    OUT = Wm2.shape[0]
    CB = 2 * B
    f32 = jnp.float32

    f1t = jnp.transpose(feat1, (0, 2, 1))
    f2t = jnp.transpose(feat2, (0, 2, 1))
    featA = jnp.concatenate([f1t, f2t], axis=0)
    featB = jnp.concatenate([f2t, f1t], axis=0)
    x1t = jnp.transpose(pc1, (0, 2, 1))
    x2t = jnp.transpose(pc2, (0, 2, 1))
    qxyz = jnp.concatenate([x1t, x2t], axis=0)
    kxyz = jnp.concatenate([x2t, x1t], axis=0)
    qxp = jnp.pad(qxyz, ((0, 0), (0, 0), (0, 13)))
    kxp = jnp.pad(kxyz, ((0, 0), (0, 0), (0, 13)))

    WposT = jnp.pad(Wpos.T, ((0, 13), (0, 0)))
    wfull = lambda shp: pl.BlockSpec(shp, lambda *a: tuple(0 for _ in shp))

    S0T = 1024
    cbN = lambda d: pl.BlockSpec((1, S0T, d), lambda cb, t: (cb, t, 0))
    qadd, ktab, qf, kf = pl.pallas_call(
        _stage0_body,
        grid=(CB, N // S0T),
        in_specs=[cbN(C), cbN(C), cbN(16), cbN(16),
                  wfull((C, M0)), wfull((1, M0)), wfull((C, M0)), wfull((1, M0)),
                  wfull((M0, DO)), wfull((1, DO)), wfull((16, M0)), wfull((1, M0))],
        out_specs=[cbN(M0), cbN(M0), cbN(DO), cbN(DO)],
        out_shape=[jax.ShapeDtypeStruct((CB, N, M0), f32),
                   jax.ShapeDtypeStruct((CB, N, M0), f32),
                   jax.ShapeDtypeStruct((CB, N, DO), f32),
                   jax.ShapeDtypeStruct((CB, N, DO), f32)],
    )(featA, featB, qxp, kxp,
      Wt11.T, bt11[None, :], Wt22.T, bt22[None, :],
      Wd.T, bd[None, :], WposT, bpos[None, :])

    qblk = lambda d: pl.BlockSpec((1, QT, d), lambda cb, t: (cb, t, 0))
    kblk = lambda d: pl.BlockSpec((1, N, d), lambda cb, t: (cb, 0, 0))
    kxpT = jnp.transpose(kxp, (0, 2, 1))
    idxg = pl.pallas_call(
        functools.partial(_stage1_body, n_keys=N),
        grid=(CB, N // QT),
        in_specs=[qblk(DO), kblk(DO), qblk(16), kblk(16),
                  pl.BlockSpec((1, 16, N), lambda cb, t: (cb, 0, 0))],
        out_specs=pl.BlockSpec((1, QT, KNN), lambda cb, t: (cb, t, 0)),
        out_shape=jax.ShapeDtypeStruct((CB, N, KNN), jnp.int32),
    )(qf, kf, qxp, kxp, kxpT)

    g = _sc_gather(ktab.reshape(CB * N, M0), idxg.reshape(CB * N * KNN))

    out = pl.pallas_call(
        _stage3_body,
        grid=(CB, N // QM),
        in_specs=[pl.BlockSpec((1, QM * KNN, M0), lambda cb, t: (cb, t, 0)),
                  pl.BlockSpec((1, QM, M0), lambda cb, t: (cb, t, 0)),
                  wfull((M0, M0)), wfull((1, M0)),
                  wfull((M0, OUT)), wfull((1, OUT))],
        out_specs=pl.BlockSpec((1, QM, OUT), lambda cb, t: (cb, t, 0)),
        out_shape=jax.ShapeDtypeStruct((CB, N, OUT), f32),
    )(g.reshape(CB, N * KNN, M0), qadd,
      Wm1.T, bm1[None, :], Wm2.T, bm2[None, :])

    feat1_new = jnp.transpose(out[:B], (0, 2, 1))
    feat2_new = jnp.transpose(out[B:], (0, 2, 1))
    return feat1_new, feat2_new


# speedup vs baseline: 18.5800x; 1.0039x over previous
import functools

import jax
import jax.numpy as jnp
from jax import lax
from jax.experimental import pallas as pl
from jax.experimental.pallas import tpu as pltpu
from jax.experimental.pallas import tpu_sc as plsc

KNN = 16
SLOPE = 0.1
QT = 512
QM = 256

_NC, _NS = 2, 16
_NW = _NC * _NS
_CHUNK = 128


def _dot(a, b):
    return lax.dot_general(a.astype(jnp.bfloat16), b.astype(jnp.bfloat16),
                           (((1,), (0,)), ((), ())),
                           preferred_element_type=jnp.float32)


def _leaky(x):
    return jnp.where(x >= 0, x, SLOPE * x)


def _stage0_body(featA, featB, qxp, kxp, Wt11T, bt11, Wt22T, bt22, WdT, bd,
                 WposT, bpos, qadd, ktab, qf, kf):
    fA = featA[0]
    fB = featB[0]
    xq = qxp[0]
    xk = kxp[0]
    qfeat = _dot(fA, Wt11T[...]) + bt11[...]
    kfeat = _dot(fB, Wt22T[...]) + bt22[...]
    qadd[0] = qfeat - _dot(xq, WposT[...]) + bpos[...]
    ktab[0] = kfeat + _dot(xk, WposT[...])
    f1 = _dot(qfeat, WdT[...]) + bd[...]
    f2 = _dot(kfeat, WdT[...]) + bd[...]
    n1 = jnp.sqrt(jnp.sum(f1 * f1, axis=1, keepdims=True))
    n2 = jnp.sqrt(jnp.sum(f2 * f2, axis=1, keepdims=True))
    qf[0] = f1 / (n1 + 1e-8)
    kf[0] = f2 / (n2 + 1e-8)


def _stage1_body(qf, kf, qxp, kxp, kxpT, idx_out, *, n_keys):
    cb = pl.program_id(0)
    a = qf[0]
    b = kf[0]
    xq = qxp[0]
    xk = kxp[0]
    xkT = kxpT[0]
    qsq = jnp.sum(xq * xq, axis=1, keepdims=True)
    ksq = jnp.sum(xkT * xkT, axis=0, keepdims=True)
    A = jnp.concatenate([-a, xq * -2.0], axis=1)
    Bm = jnp.concatenate([b, xk], axis=1)
    dist = lax.dot_general(A.astype(jnp.bfloat16), Bm.astype(jnp.bfloat16),
                           (((1,), (1,)), ((), ())),
                           preferred_element_type=jnp.float32)
    dist = dist + qsq + ksq + 1.0
    iotaf = lax.broadcasted_iota(jnp.int32, dist.shape, 1).astype(jnp.float32)
    bigf = jnp.float32(n_keys)
    inf = jnp.float32(jnp.inf)
    idxs = []
    d = dist
    for _ in range(KNN):
        m = jnp.min(d, axis=1, keepdims=True)
        cand = jnp.where(d == m, iotaf, bigf)
        iv = jnp.min(cand, axis=1, keepdims=True)
        idxs.append(iv)
        d = jnp.where(cand == iv, inf, d)
    idx_out[0] = (jnp.concatenate(idxs, axis=1).astype(jnp.int32)
                  + cb * n_keys)


def _stage3_body(g, qadd, Wm1T, bm1, Wm2T, bm2, out):
    gg = g[0]
    q = qadd[0]
    m0 = q.shape[1]
    h = gg.reshape(QM, KNN, m0) + q[:, None, :]
    h = _leaky(h).reshape(QM * KNN, m0)
    h = _leaky(_dot(h, Wm1T[...]) + bm1[...])
    h = _leaky(_dot(h, Wm2T[...]) + bm2[...])
    red = jnp.max(h.reshape(QM, KNN, out.shape[1]), axis=1)
    out[0] = jnp.transpose(red, (1, 0))


def _sc_gather(tbl, idx):
    total = idx.shape[0]
    d = tbl.shape[1]
    per_w = total // _NW
    steps = per_w // _CHUNK
    mesh = plsc.VectorSubcoreMesh(core_axis_name="c", subcore_axis_name="s")

    @functools.partial(
        pl.kernel,
        out_type=jax.ShapeDtypeStruct((total, d), jnp.float32),
        mesh=mesh,
        scratch_types=[
            pltpu.VMEM((2, _CHUNK), jnp.int32),
            pltpu.VMEM((2, _CHUNK, d), jnp.float32),
            pltpu.SemaphoreType.DMA,
            pltpu.SemaphoreType.DMA,
        ],
        compiler_params=pltpu.CompilerParams(use_tc_tiling_on_sc=False),
    )
    def gk(idx_hbm, tbl_hbm, out_hbm, idx_v, rows_v, sem0, sem1):
        wid = lax.axis_index("s") * _NC + lax.axis_index("c")
        base = wid * per_w
        sems = (sem0, sem1)

        def start(j, slot):
            off = base + j * _CHUNK
            pltpu.sync_copy(idx_hbm.at[pl.ds(off, _CHUNK)], idx_v.at[slot])
            return pltpu.async_copy(tbl_hbm.at[idx_v.at[slot]],
                                    rows_v.at[slot], sems[slot])

        def drain(j, slot, cp):
            off = base + j * _CHUNK
            cp.wait()
            pltpu.sync_copy(rows_v.at[slot], out_hbm.at[pl.ds(off, _CHUNK)])

        def step(jj, carry):
            j = jj * 2
            c0 = start(j, 0)
            c1 = start(j + 1, 1)
            drain(j, 0, c0)
            drain(j + 1, 1, c1)
            return carry

        lax.fori_loop(0, steps // 2, step, 0)

    return gk(idx, tbl)


def kernel(pc1, pc2, feat1, feat2, Wt11, bt11, Wt22, bt22, Wpos, bpos,
           Wd, bd, Wm1, bm1, Wm2, bm2):
    B = pc1.shape[0]
    N = pc1.shape[2]
    C = feat1.shape[1]
    M0 = Wt11.shape[0]
    DO = Wd.shape[0]
    OUT = Wm2.shape[0]
    CB = 2 * B
    f32 = jnp.float32

    f1t = jnp.transpose(feat1, (0, 2, 1))
    f2t = jnp.transpose(feat2, (0, 2, 1))
    featA = jnp.concatenate([f1t, f2t], axis=0)
    featB = jnp.concatenate([f2t, f1t], axis=0)
    x1t = jnp.transpose(pc1, (0, 2, 1))
    x2t = jnp.transpose(pc2, (0, 2, 1))
    qxyz = jnp.concatenate([x1t, x2t], axis=0)
    kxyz = jnp.concatenate([x2t, x1t], axis=0)
    qxp = jnp.pad(qxyz, ((0, 0), (0, 0), (0, 13)))
    kxp = jnp.pad(kxyz, ((0, 0), (0, 0), (0, 13)))

    WposT = jnp.pad(Wpos.T, ((0, 13), (0, 0)))
    wfull = lambda shp: pl.BlockSpec(shp, lambda *a: tuple(0 for _ in shp))

    S0T = 1024
    cbN = lambda d: pl.BlockSpec((1, S0T, d), lambda cb, t: (cb, t, 0))
    qadd, ktab, qf, kf = pl.pallas_call(
        _stage0_body,
        grid=(CB, N // S0T),
        in_specs=[cbN(C), cbN(C), cbN(16), cbN(16),
                  wfull((C, M0)), wfull((1, M0)), wfull((C, M0)), wfull((1, M0)),
                  wfull((M0, DO)), wfull((1, DO)), wfull((16, M0)), wfull((1, M0))],
        out_specs=[cbN(M0), cbN(M0), cbN(DO), cbN(DO)],
        out_shape=[jax.ShapeDtypeStruct((CB, N, M0), f32),
                   jax.ShapeDtypeStruct((CB, N, M0), f32),
                   jax.ShapeDtypeStruct((CB, N, DO), f32),
                   jax.ShapeDtypeStruct((CB, N, DO), f32)],
    )(featA, featB, qxp, kxp,
      Wt11.T, bt11[None, :], Wt22.T, bt22[None, :],
      Wd.T, bd[None, :], WposT, bpos[None, :])

    qblk = lambda d: pl.BlockSpec((1, QT, d), lambda cb, t: (cb, t, 0))
    kblk = lambda d: pl.BlockSpec((1, N, d), lambda cb, t: (cb, 0, 0))
    kxpT = jnp.transpose(kxp, (0, 2, 1))
    idxg = pl.pallas_call(
        functools.partial(_stage1_body, n_keys=N),
        grid=(CB, N // QT),
        in_specs=[qblk(DO), kblk(DO), qblk(16), kblk(16),
                  pl.BlockSpec((1, 16, N), lambda cb, t: (cb, 0, 0))],
        out_specs=pl.BlockSpec((1, QT, KNN), lambda cb, t: (cb, t, 0)),
        out_shape=jax.ShapeDtypeStruct((CB, N, KNN), jnp.int32),
    )(qf, kf, qxp, kxp, kxpT)

    g = _sc_gather(ktab.reshape(CB * N, M0), idxg.reshape(CB * N * KNN))

    out = pl.pallas_call(
        _stage3_body,
        grid=(CB, N // QM),
        in_specs=[pl.BlockSpec((1, QM * KNN, M0), lambda cb, t: (cb, t, 0)),
                  pl.BlockSpec((1, QM, M0), lambda cb, t: (cb, t, 0)),
                  wfull((M0, M0)), wfull((1, M0)),
                  wfull((M0, OUT)), wfull((1, OUT))],
        out_specs=pl.BlockSpec((1, OUT, QM), lambda cb, t: (cb, 0, t)),
        out_shape=jax.ShapeDtypeStruct((CB, OUT, N), f32),
    )(g.reshape(CB, N * KNN, M0), qadd,
      Wm1.T, bm1[None, :], Wm2.T, bm2[None, :])

    return out[:B], out[B:]


# speedup vs baseline: 19.2267x; 1.0348x over previous
import functools

import jax
import jax.numpy as jnp
from jax import lax
from jax.experimental import pallas as pl
from jax.experimental.pallas import tpu as pltpu
from jax.experimental.pallas import tpu_sc as plsc

KNN = 16
SLOPE = 0.1
QT = 512
QM = 256

_NC, _NS = 2, 16
_NW = _NC * _NS
_CHUNK = 128


def _dot(a, b):
    return lax.dot_general(a.astype(jnp.bfloat16), b.astype(jnp.bfloat16),
                           (((1,), (0,)), ((), ())),
                           preferred_element_type=jnp.float32)


def _leaky(x):
    return jnp.where(x >= 0, x, SLOPE * x)


def _stage0_body(featA, featB, qxp, kxp, Wt11T, bt11, Wt22T, bt22, WdT, bd,
                 WposT, bpos, qadd, ktab, qf, kf):
    fA = featA[0]
    fB = featB[0]
    xq = qxp[0]
    xk = kxp[0]
    qfeat = _dot(fA, Wt11T[...]) + bt11[...]
    kfeat = _dot(fB, Wt22T[...]) + bt22[...]
    qadd[0] = qfeat - _dot(xq, WposT[...]) + bpos[...]
    ktab[0] = kfeat + _dot(xk, WposT[...])
    f1 = _dot(qfeat, WdT[...]) + bd[...]
    f2 = _dot(kfeat, WdT[...]) + bd[...]
    n1 = jnp.sqrt(jnp.sum(f1 * f1, axis=1, keepdims=True))
    n2 = jnp.sqrt(jnp.sum(f2 * f2, axis=1, keepdims=True))
    qf[0] = f1 / (n1 + 1e-8)
    kf[0] = f2 / (n2 + 1e-8)


def _stage1_body(qf, kf, qxp, kxp, kxpT, idx_out, *, n_keys):
    cb = pl.program_id(0)
    a = qf[0]
    b = kf[0]
    xq = qxp[0]
    xk = kxp[0]
    xkT = kxpT[0]
    qsq = jnp.sum(xq * xq, axis=1, keepdims=True)
    ksq = jnp.sum(xkT * xkT, axis=0, keepdims=True)
    A = jnp.concatenate([-a, xq * -2.0], axis=1)
    Bm = jnp.concatenate([b, xk], axis=1)
    dist = lax.dot_general(A.astype(jnp.bfloat16), Bm.astype(jnp.bfloat16),
                           (((1,), (1,)), ((), ())),
                           preferred_element_type=jnp.float32)
    dist = dist + qsq + ksq + 1.0
    iotaf = lax.broadcasted_iota(jnp.int32, dist.shape, 1).astype(jnp.float32)
    bigf = jnp.float32(n_keys)
    inf = jnp.float32(jnp.inf)
    idxs = []
    d = dist
    for _ in range(KNN):
        m = jnp.min(d, axis=1, keepdims=True)
        cand = jnp.where(d == m, iotaf, bigf)
        iv = jnp.min(cand, axis=1, keepdims=True)
        idxs.append(iv)
        d = jnp.where(cand == iv, inf, d)
    idx_out[0] = (jnp.concatenate(idxs, axis=1).astype(jnp.int32)
                  + cb * n_keys)


def _stage3_body(g, qadd, Wm1T, bm1, Wm2T, bm2, out):
    gg = g[0]
    q = qadd[0]
    m0 = q.shape[1]
    h = gg.reshape(QM, KNN, m0) + q[:, None, :]
    h = _leaky(h).reshape(QM * KNN, m0)
    h = _leaky(_dot(h, Wm1T[...]) + bm1[...])
    h = _leaky(_dot(h, Wm2T[...]) + bm2[...])
    red = jnp.max(h.reshape(QM, KNN, out.shape[1]), axis=1)
    out[0] = jnp.transpose(red, (1, 0))


def _sc_gather(tbl, idx):
    total = idx.shape[0]
    d = tbl.shape[1]
    per_w = total // _NW
    steps = per_w // _CHUNK
    mesh = plsc.VectorSubcoreMesh(core_axis_name="c", subcore_axis_name="s")

    @functools.partial(
        pl.kernel,
        out_type=jax.ShapeDtypeStruct((total, d), jnp.float32),
        mesh=mesh,
        scratch_types=[
            pltpu.VMEM((2, _CHUNK), jnp.int32),
            pltpu.VMEM((2, _CHUNK, d), jnp.float32),
            pltpu.SemaphoreType.DMA,
            pltpu.SemaphoreType.DMA,
        ],
        compiler_params=pltpu.CompilerParams(use_tc_tiling_on_sc=False),
    )
    def gk(idx_hbm, tbl_hbm, out_hbm, idx_v, rows_v, sem0, sem1):
        wid = lax.axis_index("s") * _NC + lax.axis_index("c")
        base = wid * per_w
        sems = (sem0, sem1)

        def start(j, slot):
            off = base + j * _CHUNK
            pltpu.sync_copy(idx_hbm.at[pl.ds(off, _CHUNK)], idx_v.at[slot])
            return pltpu.async_copy(tbl_hbm.at[idx_v.at[slot]],
                                    rows_v.at[slot], sems[slot])

        def drain(j, slot, cp):
            off = base + j * _CHUNK
            cp.wait()
            pltpu.sync_copy(rows_v.at[slot], out_hbm.at[pl.ds(off, _CHUNK)])

        def step(jj, carry):
            j = jj * 2
            c0 = start(j, 0)
            c1 = start(j + 1, 1)
            drain(j, 0, c0)
            drain(j + 1, 1, c1)
            return carry

        lax.fori_loop(0, steps // 2, step, 0)

    return gk(idx, tbl)


def kernel(pc1, pc2, feat1, feat2, Wt11, bt11, Wt22, bt22, Wpos, bpos,
           Wd, bd, Wm1, bm1, Wm2, bm2):
    B = pc1.shape[0]
    N = pc1.shape[2]
    C = feat1.shape[1]
    M0 = Wt11.shape[0]
    DO = Wd.shape[0]
    OUT = Wm2.shape[0]
    CB = 2 * B
    f32 = jnp.float32

    f1t = jnp.transpose(feat1, (0, 2, 1))
    f2t = jnp.transpose(feat2, (0, 2, 1))
    featA = jnp.concatenate([f1t, f2t], axis=0)
    featB = jnp.concatenate([f2t, f1t], axis=0)
    x1t = jnp.transpose(pc1, (0, 2, 1))
    x2t = jnp.transpose(pc2, (0, 2, 1))
    qxyz = jnp.concatenate([x1t, x2t], axis=0)
    kxyz = jnp.concatenate([x2t, x1t], axis=0)
    qxp = jnp.pad(qxyz, ((0, 0), (0, 0), (0, 13)))
    kxp = jnp.pad(kxyz, ((0, 0), (0, 0), (0, 13)))

    WposT = jnp.pad(Wpos.T, ((0, 13), (0, 0)))
    wfull = lambda shp: pl.BlockSpec(shp, lambda *a: tuple(0 for _ in shp))

    S0T = 1024
    cbN = lambda d: pl.BlockSpec((1, S0T, d), lambda cb, t: (cb, t, 0))
    qadd, ktab, qf, kf = pl.pallas_call(
        _stage0_body,
        grid=(CB, N // S0T),
        in_specs=[cbN(C), cbN(C), cbN(16), cbN(16),
                  wfull((C, M0)), wfull((1, M0)), wfull((C, M0)), wfull((1, M0)),
                  wfull((M0, DO)), wfull((1, DO)), wfull((16, M0)), wfull((1, M0))],
        out_specs=[cbN(M0), cbN(M0), cbN(DO), cbN(DO)],
        out_shape=[jax.ShapeDtypeStruct((CB, N, M0), f32),
                   jax.ShapeDtypeStruct((CB, N, M0), f32),
                   jax.ShapeDtypeStruct((CB, N, DO), f32),
                   jax.ShapeDtypeStruct((CB, N, DO), f32)],
    )(featA, featB, qxp, kxp,
      Wt11.T, bt11[None, :], Wt22.T, bt22[None, :],
      Wd.T, bd[None, :], WposT, bpos[None, :])

    kxpT = jnp.transpose(kxp, (0, 2, 1))

    def half(qf_h, kf_h, qxp_h, kxp_h, kxpT_h, ktab_h, qadd_h):
        nb = qf_h.shape[0]
        qblk = lambda d: pl.BlockSpec((1, QT, d), lambda cb, t: (cb, t, 0))
        kblk = lambda d: pl.BlockSpec((1, N, d), lambda cb, t: (cb, 0, 0))
        idxg = pl.pallas_call(
            functools.partial(_stage1_body, n_keys=N),
            grid=(nb, N // QT),
            in_specs=[qblk(DO), kblk(DO), qblk(16), kblk(16),
                      pl.BlockSpec((1, 16, N), lambda cb, t: (cb, 0, 0))],
            out_specs=pl.BlockSpec((1, QT, KNN), lambda cb, t: (cb, t, 0)),
            out_shape=jax.ShapeDtypeStruct((nb, N, KNN), jnp.int32),
        )(qf_h, kf_h, qxp_h, kxp_h, kxpT_h)

        g = _sc_gather(ktab_h.reshape(nb * N, M0), idxg.reshape(nb * N * KNN))

        return pl.pallas_call(
            _stage3_body,
            grid=(nb, N // QM),
            in_specs=[pl.BlockSpec((1, QM * KNN, M0), lambda cb, t: (cb, t, 0)),
                      pl.BlockSpec((1, QM, M0), lambda cb, t: (cb, t, 0)),
                      wfull((M0, M0)), wfull((1, M0)),
                      wfull((M0, OUT)), wfull((1, OUT))],
            out_specs=pl.BlockSpec((1, OUT, QM), lambda cb, t: (cb, 0, t)),
            out_shape=jax.ShapeDtypeStruct((nb, OUT, N), f32),
        )(g.reshape(nb, N * KNN, M0), qadd_h,
          Wm1.T, bm1[None, :], Wm2.T, bm2[None, :])

    feat1_new = half(qf[:B], kf[:B], qxp[:B], kxp[:B], kxpT[:B],
                     ktab[:B], qadd[:B])
    feat2_new = half(qf[B:], kf[B:], qxp[B:], kxp[B:], kxpT[B:],
                     ktab[B:], qadd[B:])
    return feat1_new, feat2_new
